# baseline (device time: 65186 ns/iter reference)
import jax
import jax.numpy as jnp
from jax import lax
from jax.experimental import pallas as pl
from jax.experimental.pallas import tpu as pltpu

B, S, HL, D = 2, 1024, 16, 64
K = HL * D
N = 2048
S_HALF = S // 2
CH = 4
SC = S_HALF // CH
NCHUNK = B * CH


def kernel(O, Wo):
    OT = O.transpose(0, 2, 3, 1).reshape(B, K, S)

    def body(ot_ref, w_ref, out_ref, send_buf, recv_buf,
             send_sems, recv_sems):
        my_x = lax.axis_index("x")
        my_y = lax.axis_index("y")
        my_z = lax.axis_index("z")
        partner = 1 - my_x

        barrier_sem = pltpu.get_barrier_semaphore()
        pl.semaphore_signal(
            barrier_sem, inc=1,
            device_id=(partner, my_y, my_z),
            device_id_type=pl.DeviceIdType.MESH,
        )
        pl.semaphore_wait(barrier_sem, 1)

        w16 = w_ref[...].astype(jnp.bfloat16)

        rdmas = []
        for b in range(B):
            for c in range(CH):
                i = b * CH + c
                lhs = ot_ref[
                    b, :, pl.ds(partner * S_HALF + c * SC, SC)
                ].astype(jnp.bfloat16)
                acc = lax.dot_general(
                    lhs, w16, (((0,), (0,)), ((), ())),
                    preferred_element_type=jnp.float32,
                )
                send_buf[b, c * SC:(c + 1) * SC] = acc.astype(jnp.bfloat16)
                rdma = pltpu.make_async_remote_copy(
                    src_ref=send_buf.at[b, pl.ds(c * SC, SC)],
                    dst_ref=recv_buf.at[b, pl.ds(c * SC, SC)],
                    send_sem=send_sems.at[i],
                    recv_sem=recv_sems.at[i],
                    device_id=(partner, my_y, my_z),
                    device_id_type=pl.DeviceIdType.MESH,
                )
                rdma.start()
                rdmas.append(rdma)

        for b in range(B):
            lhs = ot_ref[b, :, pl.ds(my_x * S_HALF, S_HALF)].astype(jnp.bfloat16)
            acc = lax.dot_general(
                lhs, w16, (((0,), (0,)), ((), ())),
                preferred_element_type=jnp.float32,
            )
            out_ref[b] = acc

        for b in range(B):
            for c in range(CH):
                i = b * CH + c
                rdmas[i].wait_recv()
                sl = slice(c * SC, (c + 1) * SC)
                out_ref[b, sl] += recv_buf[b, sl].astype(jnp.float32)

        for rdma in rdmas:
            rdma.wait_send()

    return pl.pallas_call(
        body,
        out_shape=jax.ShapeDtypeStruct((B, S_HALF, N), jnp.float32),
        in_specs=[
            pl.BlockSpec(memory_space=pltpu.VMEM),
            pl.BlockSpec(memory_space=pltpu.VMEM),
        ],
        out_specs=pl.BlockSpec(memory_space=pltpu.VMEM),
        scratch_shapes=[
            pltpu.VMEM((B, S_HALF, N), jnp.bfloat16),
            pltpu.VMEM((B, S_HALF, N), jnp.bfloat16),
            pltpu.SemaphoreType.DMA((NCHUNK,)),
            pltpu.SemaphoreType.DMA((NCHUNK,)),
        ],
        compiler_params=pltpu.CompilerParams(
            collective_id=0,
            vmem_limit_bytes=100 * 1024 * 1024,
        ),
    )(OT, Wo)


# device time: 64555 ns/iter; 1.0098x vs baseline; 1.0098x over previous
import jax
import jax.numpy as jnp
from jax import lax
from jax.experimental import pallas as pl
from jax.experimental.pallas import tpu as pltpu

B, S, HL, D = 2, 1024, 16, 64
K = HL * D
N = 2048
NH = N // 2
S_HALF = S // 2
CH = 4
SC = S_HALF // CH
NSEND = B * CH + 1


def kernel(O, Wo):
    OT = O.transpose(0, 2, 3, 1).reshape(B, K, S)

    def body(ot_hbm, w_hbm, out_ref, ot_vmem, wf_vmem, w16_vmem,
             send_buf, recv_buf, send_sems, recv_sems, oload_sems,
             own_sems, wload_sem):
        my_x = lax.axis_index("x")
        my_y = lax.axis_index("y")
        my_z = lax.axis_index("z")
        partner = 1 - my_x

        barrier_sem = pltpu.get_barrier_semaphore()
        pl.semaphore_signal(
            barrier_sem, inc=1,
            device_id=(partner, my_y, my_z),
            device_id_type=pl.DeviceIdType.MESH,
        )

        wload = pltpu.make_async_copy(w_hbm, wf_vmem, wload_sem)
        wload.start()
        oloads, ownloads = [], []
        for b in range(B):
            for c in range(CH):
                sl = pl.ds(partner * S_HALF + c * SC, SC)
                cp = pltpu.make_async_copy(
                    ot_hbm.at[b, :, sl], ot_vmem.at[b, :, sl],
                    oload_sems.at[b * CH + c],
                )
                cp.start()
                oloads.append(cp)
        for b in range(B):
            sl = pl.ds(my_x * S_HALF, S_HALF)
            cp = pltpu.make_async_copy(
                ot_hbm.at[b, :, sl], ot_vmem.at[b, :, sl], own_sems.at[b]
            )
            cp.start()
            ownloads.append(cp)

        pl.semaphore_wait(barrier_sem, 1)
        wload.wait()

        def tdot(lhs, rhs):
            return lax.dot_general(
                lhs, rhs, (((0,), (0,)), ((), ())),
                preferred_element_type=jnp.float32,
            )

        rdmas = []
        oloads[0].wait()
        lhs0 = ot_vmem[0, :, pl.ds(partner * S_HALF, SC)].astype(jnp.bfloat16)
        for h in range(2):
            nsl = slice(h * NH, (h + 1) * NH)
            w16_vmem[:, nsl] = wf_vmem[:, nsl].astype(jnp.bfloat16)
            send_buf[0, :SC, nsl] = tdot(lhs0, w16_vmem[:, nsl]).astype(
                jnp.bfloat16
            )
            rdma = pltpu.make_async_remote_copy(
                src_ref=send_buf.at[0, pl.ds(0, SC), nsl],
                dst_ref=recv_buf.at[0, pl.ds(0, SC), nsl],
                send_sem=send_sems.at[h],
                recv_sem=recv_sems.at[h],
                device_id=(partner, my_y, my_z),
                device_id_type=pl.DeviceIdType.MESH,
            )
            rdma.start()
            rdmas.append(rdma)

        for b in range(B):
            for c in range(CH):
                if b == 0 and c == 0:
                    continue
                i = b * CH + c + 1
                oloads[b * CH + c].wait()
                lhs = ot_vmem[
                    b, :, pl.ds(partner * S_HALF + c * SC, SC)
                ].astype(jnp.bfloat16)
                send_buf[b, c * SC:(c + 1) * SC] = tdot(
                    lhs, w16_vmem[...]
                ).astype(jnp.bfloat16)
                rdma = pltpu.make_async_remote_copy(
                    src_ref=send_buf.at[b, pl.ds(c * SC, SC)],
                    dst_ref=recv_buf.at[b, pl.ds(c * SC, SC)],
                    send_sem=send_sems.at[i],
                    recv_sem=recv_sems.at[i],
                    device_id=(partner, my_y, my_z),
                    device_id_type=pl.DeviceIdType.MESH,
                )
                rdma.start()
                rdmas.append(rdma)

        for b in range(B):
            ownloads[b].wait()
            lhs = ot_vmem[b, :, pl.ds(my_x * S_HALF, S_HALF)].astype(
                jnp.bfloat16
            )
            out_ref[b] = tdot(lhs, w16_vmem[...])

        rdmas[0].wait_recv()
        rdmas[1].wait_recv()
        out_ref[0, :SC] += recv_buf[0, :SC].astype(jnp.float32)
        for b in range(B):
            for c in range(CH):
                if b == 0 and c == 0:
                    continue
                rdmas[b * CH + c + 1].wait_recv()
                sl = slice(c * SC, (c + 1) * SC)
                out_ref[b, sl] += recv_buf[b, sl].astype(jnp.float32)

        for rdma in rdmas:
            rdma.wait_send()

    return pl.pallas_call(
        body,
        out_shape=jax.ShapeDtypeStruct((B, S_HALF, N), jnp.float32),
        in_specs=[
            pl.BlockSpec(memory_space=pl.ANY),
            pl.BlockSpec(memory_space=pl.ANY),
        ],
        out_specs=pl.BlockSpec(memory_space=pltpu.VMEM),
        scratch_shapes=[
            pltpu.VMEM((B, K, S), jnp.float32),
            pltpu.VMEM((K, N), jnp.float32),
            pltpu.VMEM((K, N), jnp.bfloat16),
            pltpu.VMEM((B, S_HALF, N), jnp.bfloat16),
            pltpu.VMEM((B, S_HALF, N), jnp.bfloat16),
            pltpu.SemaphoreType.DMA((NSEND,)),
            pltpu.SemaphoreType.DMA((NSEND,)),
            pltpu.SemaphoreType.DMA((B * CH,)),
            pltpu.SemaphoreType.DMA((B,)),
            pltpu.SemaphoreType.DMA,
        ],
        compiler_params=pltpu.CompilerParams(
            collective_id=0,
            vmem_limit_bytes=100 * 1024 * 1024,
        ),
    )(OT, Wo)


# device time: 61461 ns/iter; 1.0606x vs baseline; 1.0503x over previous
import jax
import jax.numpy as jnp
from jax import lax
from jax.experimental import pallas as pl
from jax.experimental.pallas import tpu as pltpu

B, S, HL, D = 2, 1024, 16, 64
K = HL * D
N = 2048
NH = N // 2
S_HALF = S // 2
CH = 4
SC = S_HALF // CH
NSEND = B * CH + 1


def kernel(O, Wo):
    OT = O.transpose(0, 2, 3, 1).reshape(B, K, S)

    def body(ot_hbm, w_hbm, out_ref, ot_vmem, wf_vmem, w16_vmem, acc_vmem,
             send_buf, recv_buf, send_sems, recv_sems, oload_sems,
             own_sems, wload_sem):
        my_x = lax.axis_index("x")
        my_y = lax.axis_index("y")
        my_z = lax.axis_index("z")
        partner = 1 - my_x

        barrier_sem = pltpu.get_barrier_semaphore()
        pl.semaphore_signal(
            barrier_sem, inc=1,
            device_id=(partner, my_y, my_z),
            device_id_type=pl.DeviceIdType.MESH,
        )

        wload = pltpu.make_async_copy(w_hbm, wf_vmem, wload_sem)
        wload.start()
        oloads, ownloads = [], []
        for b in range(B):
            for c in range(CH):
                sl = pl.ds(partner * S_HALF + c * SC, SC)
                cp = pltpu.make_async_copy(
                    ot_hbm.at[b, :, sl], ot_vmem.at[b, :, sl],
                    oload_sems.at[b * CH + c],
                )
                cp.start()
                oloads.append(cp)
        for b in range(B):
            sl = pl.ds(my_x * S_HALF, S_HALF)
            cp = pltpu.make_async_copy(
                ot_hbm.at[b, :, sl], ot_vmem.at[b, :, sl], own_sems.at[b]
            )
            cp.start()
            ownloads.append(cp)

        pl.semaphore_wait(barrier_sem, 1)
        wload.wait()

        def tdot(lhs, rhs):
            return lax.dot_general(
                lhs, rhs, (((0,), (0,)), ((), ())),
                preferred_element_type=jnp.float32,
            )

        rdmas = []
        oloads[0].wait()
        lhs0 = ot_vmem[0, :, pl.ds(partner * S_HALF, SC)].astype(jnp.bfloat16)
        for h in range(2):
            nsl = slice(h * NH, (h + 1) * NH)
            w16_vmem[:, nsl] = wf_vmem[:, nsl].astype(jnp.bfloat16)
            send_buf[0, :SC, nsl] = tdot(lhs0, w16_vmem[:, nsl]).astype(
                jnp.bfloat16
            )
            rdma = pltpu.make_async_remote_copy(
                src_ref=send_buf.at[0, pl.ds(0, SC), nsl],
                dst_ref=recv_buf.at[0, pl.ds(0, SC), nsl],
                send_sem=send_sems.at[h],
                recv_sem=recv_sems.at[h],
                device_id=(partner, my_y, my_z),
                device_id_type=pl.DeviceIdType.MESH,
            )
            rdma.start()
            rdmas.append(rdma)

        for b in range(B):
            for c in range(CH):
                if b == 0 and c == 0:
                    continue
                i = b * CH + c + 1
                oloads[b * CH + c].wait()
                lhs = ot_vmem[
                    b, :, pl.ds(partner * S_HALF + c * SC, SC)
                ].astype(jnp.bfloat16)
                send_buf[b, c * SC:(c + 1) * SC] = tdot(
                    lhs, w16_vmem[...]
                ).astype(jnp.bfloat16)
                rdma = pltpu.make_async_remote_copy(
                    src_ref=send_buf.at[b, pl.ds(c * SC, SC)],
                    dst_ref=recv_buf.at[b, pl.ds(c * SC, SC)],
                    send_sem=send_sems.at[i],
                    recv_sem=recv_sems.at[i],
                    device_id=(partner, my_y, my_z),
                    device_id_type=pl.DeviceIdType.MESH,
                )
                rdma.start()
                rdmas.append(rdma)

        for b in range(B):
            ownloads[b].wait()
            lhs = ot_vmem[b, :, pl.ds(my_x * S_HALF, S_HALF)].astype(
                jnp.bfloat16
            )
            acc_vmem[b] = tdot(lhs, w16_vmem[...])

        rdmas[0].wait_recv()
        rdmas[1].wait_recv()
        out_ref[0, :SC] = (
            acc_vmem[0, :SC] + recv_buf[0, :SC].astype(jnp.float32)
        ).astype(jnp.bfloat16)
        for b in range(B):
            for c in range(CH):
                if b == 0 and c == 0:
                    continue
                rdmas[b * CH + c + 1].wait_recv()
                sl = slice(c * SC, (c + 1) * SC)
                out_ref[b, sl] = (
                    acc_vmem[b, sl] + recv_buf[b, sl].astype(jnp.float32)
                ).astype(jnp.bfloat16)

        for rdma in rdmas:
            rdma.wait_send()

    return pl.pallas_call(
        body,
        out_shape=jax.ShapeDtypeStruct((B, S_HALF, N), jnp.bfloat16),
        in_specs=[
            pl.BlockSpec(memory_space=pl.ANY),
            pl.BlockSpec(memory_space=pl.ANY),
        ],
        out_specs=pl.BlockSpec(memory_space=pltpu.VMEM),
        scratch_shapes=[
            pltpu.VMEM((B, K, S), jnp.float32),
            pltpu.VMEM((K, N), jnp.float32),
            pltpu.VMEM((K, N), jnp.bfloat16),
            pltpu.VMEM((B, S_HALF, N), jnp.float32),
            pltpu.VMEM((B, S_HALF, N), jnp.bfloat16),
            pltpu.VMEM((B, S_HALF, N), jnp.bfloat16),
            pltpu.SemaphoreType.DMA((NSEND,)),
            pltpu.SemaphoreType.DMA((NSEND,)),
            pltpu.SemaphoreType.DMA((B * CH,)),
            pltpu.SemaphoreType.DMA((B,)),
            pltpu.SemaphoreType.DMA,
        ],
        compiler_params=pltpu.CompilerParams(
            collective_id=0,
            vmem_limit_bytes=100 * 1024 * 1024,
        ),
    )(OT, Wo)


# device time: 60827 ns/iter; 1.0717x vs baseline; 1.0104x over previous
import jax
import jax.numpy as jnp
from jax import lax
from jax.experimental import pallas as pl
from jax.experimental.pallas import tpu as pltpu

B, S, HL, D = 2, 1024, 16, 64
K = HL * D
N = 2048
NH = N // 2
S_HALF = S // 2
CH = 4
SC = S_HALF // CH
NSEND = B * CH + 1


def kernel(O, Wo):
    OT = O.transpose(0, 2, 3, 1).reshape(B, K, S)

    def body(ot_hbm, w_hbm, out_ref, ot_vmem, wf_vmem, w16_vmem, acc_vmem,
             send_buf, recv_buf, send_sems, recv_sems, oload_sems,
             wload_sem):
        my_x = lax.axis_index("x")
        my_y = lax.axis_index("y")
        my_z = lax.axis_index("z")
        partner = 1 - my_x

        barrier_sem = pltpu.get_barrier_semaphore()
        pl.semaphore_signal(
            barrier_sem, inc=1,
            device_id=(partner, my_y, my_z),
            device_id_type=pl.DeviceIdType.MESH,
        )

        wload = pltpu.make_async_copy(w_hbm, wf_vmem, wload_sem)
        wload.start()
        oloads = []
        for b in range(B):
            cp = pltpu.make_async_copy(
                ot_hbm.at[b], ot_vmem.at[b], oload_sems.at[b]
            )
            cp.start()
            oloads.append(cp)

        pl.semaphore_wait(barrier_sem, 1)
        wload.wait()

        def tdot(lhs, rhs):
            return lax.dot_general(
                lhs, rhs, (((0,), (0,)), ((), ())),
                preferred_element_type=jnp.float32,
            )

        rdmas = []
        oloads[0].wait()
        lhs0 = ot_vmem[0, :, pl.ds(partner * S_HALF, SC)].astype(jnp.bfloat16)
        for h in range(2):
            nsl = slice(h * NH, (h + 1) * NH)
            w16_vmem[:, nsl] = wf_vmem[:, nsl].astype(jnp.bfloat16)
            send_buf[0, :SC, nsl] = tdot(lhs0, w16_vmem[:, nsl]).astype(
                jnp.bfloat16
            )
            rdma = pltpu.make_async_remote_copy(
                src_ref=send_buf.at[0, pl.ds(0, SC), nsl],
                dst_ref=recv_buf.at[0, pl.ds(0, SC), nsl],
                send_sem=send_sems.at[h],
                recv_sem=recv_sems.at[h],
                device_id=(partner, my_y, my_z),
                device_id_type=pl.DeviceIdType.MESH,
            )
            rdma.start()
            rdmas.append(rdma)

        for b in range(B):
            for c in range(CH):
                if b == 0 and c == 0:
                    continue
                i = b * CH + c + 1
                if c == 0:
                    oloads[b].wait()
                lhs = ot_vmem[
                    b, :, pl.ds(partner * S_HALF + c * SC, SC)
                ].astype(jnp.bfloat16)
                send_buf[b, c * SC:(c + 1) * SC] = tdot(
                    lhs, w16_vmem[...]
                ).astype(jnp.bfloat16)
                rdma = pltpu.make_async_remote_copy(
                    src_ref=send_buf.at[b, pl.ds(c * SC, SC)],
                    dst_ref=recv_buf.at[b, pl.ds(c * SC, SC)],
                    send_sem=send_sems.at[i],
                    recv_sem=recv_sems.at[i],
                    device_id=(partner, my_y, my_z),
                    device_id_type=pl.DeviceIdType.MESH,
                )
                rdma.start()
                rdmas.append(rdma)

        for b in range(B):
            lhs = ot_vmem[b, :, pl.ds(my_x * S_HALF, S_HALF)].astype(
                jnp.bfloat16
            )
            acc_vmem[b] = tdot(lhs, w16_vmem[...])

        rdmas[0].wait_recv()
        rdmas[1].wait_recv()
        out_ref[0, :SC] = (
            acc_vmem[0, :SC] + recv_buf[0, :SC].astype(jnp.float32)
        ).astype(jnp.bfloat16)
        for b in range(B):
            for c in range(CH):
                if b == 0 and c == 0:
                    continue
                rdmas[b * CH + c + 1].wait_recv()
                sl = slice(c * SC, (c + 1) * SC)
                out_ref[b, sl] = (
                    acc_vmem[b, sl] + recv_buf[b, sl].astype(jnp.float32)
                ).astype(jnp.bfloat16)

        for rdma in rdmas:
            rdma.wait_send()

    return pl.pallas_call(
        body,
        out_shape=jax.ShapeDtypeStruct((B, S_HALF, N), jnp.bfloat16),
        in_specs=[
            pl.BlockSpec(memory_space=pl.ANY),
            pl.BlockSpec(memory_space=pl.ANY),
        ],
        out_specs=pl.BlockSpec(memory_space=pltpu.VMEM),
        scratch_shapes=[
            pltpu.VMEM((B, K, S), jnp.float32),
            pltpu.VMEM((K, N), jnp.float32),
            pltpu.VMEM((K, N), jnp.bfloat16),
            pltpu.VMEM((B, S_HALF, N), jnp.float32),
            pltpu.VMEM((B, S_HALF, N), jnp.bfloat16),
            pltpu.VMEM((B, S_HALF, N), jnp.bfloat16),
            pltpu.SemaphoreType.DMA((NSEND,)),
            pltpu.SemaphoreType.DMA((NSEND,)),
            pltpu.SemaphoreType.DMA((B,)),
            pltpu.SemaphoreType.DMA,
        ],
        compiler_params=pltpu.CompilerParams(
            collective_id=0,
            vmem_limit_bytes=100 * 1024 * 1024,
        ),
    )(OT, Wo)


# device time: 59399 ns/iter; 1.0974x vs baseline; 1.0240x over previous
import jax
import jax.numpy as jnp
from jax import lax
from jax.experimental import pallas as pl
from jax.experimental.pallas import tpu as pltpu

B, S, HL, D = 2, 1024, 16, 64
K = HL * D
N = 2048
NH = N // 2
S_HALF = S // 2

OPS = (
    [(0, 0, 128, 0, NH // 2), (0, 0, 128, NH // 2, NH // 2),
     (0, 0, 128, NH, NH)]
    + [(0, r, 128, 0, N) for r in (128, 256, 384)]
    + [(1, r, 128, 0, N) for r in (0, 128, 256, 384)]
)
NSEND = len(OPS)


def kernel(O, Wo):
    OT = O.transpose(0, 2, 3, 1).reshape(B, K, S)

    def body(ot_hbm, w_hbm, out_ref, ot_vmem, wf_vmem, w16_vmem, acc_vmem,
             send_buf, recv_buf, send_sems, recv_sems, oload_sems,
             wload_sems):
        my_x = lax.axis_index("x")
        my_y = lax.axis_index("y")
        my_z = lax.axis_index("z")
        partner = 1 - my_x

        barrier_sem = pltpu.get_barrier_semaphore()
        pl.semaphore_signal(
            barrier_sem, inc=1,
            device_id=(partner, my_y, my_z),
            device_id_type=pl.DeviceIdType.MESH,
        )

        wloads = [
            pltpu.make_async_copy(
                w_hbm.at[:, h * NH:(h + 1) * NH],
                wf_vmem.at[:, h * NH:(h + 1) * NH],
                wload_sems.at[h],
            )
            for h in range(2)
        ]
        oloads = [
            pltpu.make_async_copy(
                ot_hbm.at[b], ot_vmem.at[b], oload_sems.at[b]
            )
            for b in range(B)
        ]
        wloads[0].start()
        oloads[0].start()
        wloads[1].start()
        oloads[1].start()

        def tdot(lhs, rhs):
            return lax.dot_general(
                lhs, rhs, (((0,), (0,)), ((), ())),
                preferred_element_type=jnp.float32,
            )

        w_ready = [False, False]
        o_ready = [False, False]
        barrier_done = False

        rdmas = []
        for idx, (b, r0, rl, n0, nl) in enumerate(OPS):
            for h in range(2):
                if not w_ready[h] and n0 <= h * NH < n0 + nl:
                    wloads[h].wait()
                    sl = slice(h * NH, (h + 1) * NH)
                    w16_vmem[:, sl] = wf_vmem[:, sl].astype(jnp.bfloat16)
                    w_ready[h] = True
            if not o_ready[b]:
                oloads[b].wait()
                o_ready[b] = True
            lhs = ot_vmem[
                b, :, pl.ds(partner * S_HALF + r0, rl)
            ].astype(jnp.bfloat16)
            rsl = slice(r0, r0 + rl)
            nsl = slice(n0, n0 + nl)
            send_buf[b, rsl, nsl] = tdot(lhs, w16_vmem[:, nsl]).astype(
                jnp.bfloat16
            )
            if not barrier_done:
                pl.semaphore_wait(barrier_sem, 1)
                barrier_done = True
            rdma = pltpu.make_async_remote_copy(
                src_ref=send_buf.at[b, rsl, nsl],
                dst_ref=recv_buf.at[b, rsl, nsl],
                send_sem=send_sems.at[idx],
                recv_sem=recv_sems.at[idx],
                device_id=(partner, my_y, my_z),
                device_id_type=pl.DeviceIdType.MESH,
            )
            rdma.start()
            rdmas.append(rdma)

        for b in range(B):
            lhs = ot_vmem[b, :, pl.ds(my_x * S_HALF, S_HALF)].astype(
                jnp.bfloat16
            )
            acc_vmem[b] = tdot(lhs, w16_vmem[...])

        for idx, (b, r0, rl, n0, nl) in enumerate(OPS):
            rdmas[idx].wait_recv()
            rsl = slice(r0, r0 + rl)
            nsl = slice(n0, n0 + nl)
            out_ref[b, rsl, nsl] = (
                acc_vmem[b, rsl, nsl]
                + recv_buf[b, rsl, nsl].astype(jnp.float32)
            ).astype(jnp.bfloat16)

        for rdma in rdmas:
            rdma.wait_send()

    return pl.pallas_call(
        body,
        out_shape=jax.ShapeDtypeStruct((B, S_HALF, N), jnp.bfloat16),
        in_specs=[
            pl.BlockSpec(memory_space=pl.ANY),
            pl.BlockSpec(memory_space=pl.ANY),
        ],
        out_specs=pl.BlockSpec(memory_space=pltpu.VMEM),
        scratch_shapes=[
            pltpu.VMEM((B, K, S), jnp.float32),
            pltpu.VMEM((K, N), jnp.float32),
            pltpu.VMEM((K, N), jnp.bfloat16),
            pltpu.VMEM((B, S_HALF, N), jnp.float32),
            pltpu.VMEM((B, S_HALF, N), jnp.bfloat16),
            pltpu.VMEM((B, S_HALF, N), jnp.bfloat16),
            pltpu.SemaphoreType.DMA((NSEND,)),
            pltpu.SemaphoreType.DMA((NSEND,)),
            pltpu.SemaphoreType.DMA((B,)),
            pltpu.SemaphoreType.DMA((2,)),
        ],
        compiler_params=pltpu.CompilerParams(
            collective_id=0,
            vmem_limit_bytes=100 * 1024 * 1024,
        ),
    )(OT, Wo)
